# baseline (speedup 1.0000x reference)
import jax
import jax.numpy as jnp
from jax.experimental import pallas as pl


def _mm_kernel(x_ref, cb_ref, o_ref):
    o_ref[...] = jnp.dot(x_ref[...], cb_ref[...],
                         preferred_element_type=jnp.float32)


def kernel(x, codebook):
    B, N, K = x.shape
    D = codebook.shape[1]
    M = B * N
    xm = x.reshape(M, K)
    bm = 512
    out = pl.pallas_call(
        _mm_kernel,
        grid=(M // bm,),
        in_specs=[
            pl.BlockSpec((bm, K), lambda i: (i, 0)),
            pl.BlockSpec((K, D), lambda i: (0, 0)),
        ],
        out_specs=pl.BlockSpec((bm, D), lambda i: (i, 0)),
        out_shape=jax.ShapeDtypeStruct((M, D), jnp.float32),
    )(xm, codebook)
    return out.reshape(B, N, D)
